# Initial kernel scaffold; baseline (speedup 1.0000x reference)
import jax
import jax.numpy as jnp
from jax.experimental import pallas as pl


def kernel(legal_mask, current_player, current_hash, ZposT, cap_indptr, cap_indices, can_capture_any, hash_history):
    raise NotImplementedError("write your pallas kernel here")


# speedup vs baseline: 237.2480x; 237.2480x over previous
import jax
import jax.numpy as jnp
from jax import lax
from jax.experimental import pallas as pl
from jax.experimental.pallas import tpu as pltpu
from jax.experimental.pallas import tpu_sc as plsc

_B = 4096
_N2 = 361
_ROWS = _B * _N2
_L = 2 * _ROWS
_NW = 32
_NR = _ROWS // _NW
_BPT = _B // _NW
_NCH = 8
_BPC = 16
_RPC = _BPC * _N2
_RPAD = 5888
_NV = _RPAD // 16
_CE = 8192
_NVW = _CE // 16


def _sc_body(ip_hbm, ci_hbm, leg_hbm, cc_hbm, hist_hbm, pl_hbm, hash_hbm,
             d0_hbm, d1_hbm, ptab_hbm,
             nh_hbm, filt_hbm, p0_hbm, p1_hbm,
             ip_v, ipm1_v, g0_v, g1_v, lo_v, hi_v,
             leg_v, cc_v, nh_v, ch_v, d0_v, d1_v, ptab_v,
             pof_v, hsh_v, tst_v, d0s, d1s, hists, sem):
    sid = lax.axis_index("s")
    wid = sid * 2 + lax.axis_index("c")
    tr0 = wid * _NR
    tb0 = wid * _BPT
    lanes = lax.iota(jnp.int32, 16)
    zero16 = jnp.zeros((16,), jnp.int32)

    pltpu.sync_copy(ptab_hbm, ptab_v)

    @pl.when(sid == 0)
    def _():
        pltpu.sync_copy(d0_hbm, d0s)
        pltpu.sync_copy(d1_hbm, d1s)

    pltpu.sync_copy(hist_hbm.at[pl.ds(tb0 * _N2, _NR)],
                    hists.at[pl.ds(sid * _NR, _NR)])
    plsc.subcore_barrier()
    pltpu.sync_copy(ip_hbm.at[pl.ds(tr0, 16)], tst_v)
    e0 = tst_v[pl.ds(0, 16)][0]
    pltpu.sync_copy(ip_hbm.at[pl.ds(tr0 + _NR, 16)], tst_v)
    e1 = tst_v[pl.ds(0, 16)][0]
    e8 = e0 & jnp.int32(~127)
    n_win = lax.div(e1 - e8 + jnp.int32(_CE - 1), jnp.int32(_CE))
    pbase = wid * (_L + 2 * _CE)

    def window(i, carry):
        c0, c1 = carry
        base = pl.multiple_of(e8 + i * _CE, 128)
        pltpu.sync_copy(ci_hbm.at[pl.ds(base, _CE)], ch_v)
        cp0 = pltpu.async_copy(d0s.at[ch_v], d0_v, sem)
        cp1 = pltpu.async_copy(d1s.at[ch_v], d1_v, sem)
        cp0.wait()
        cp1.wait()

        def scan_vreg(j, carry):
            c0, c1 = carry
            o = j * 16
            own = (base + o + lanes) >= e0
            v0 = jnp.where(own, d0_v[pl.ds(o, 16)], 0)
            v1 = jnp.where(own, d1_v[pl.ds(o, 16)], 0)
            for k in (1, 2, 4, 8):
                src = jnp.maximum(lanes - k, 0)
                m = lanes >= k
                v0 = v0 ^ jnp.where(m, v0.at[src].get(mode="promise_in_bounds"), 0)
                v1 = v1 ^ jnp.where(m, v1.at[src].get(mode="promise_in_bounds"), 0)
            v0 = v0 ^ c0
            v1 = v1 ^ c1
            d0_v[pl.ds(o, 16)] = v0
            d1_v[pl.ds(o, 16)] = v1
            c0 = jnp.full((16,), v0[15], jnp.int32)
            c1 = jnp.full((16,), v1[15], jnp.int32)
            return c0, c1

        c0, c1 = lax.fori_loop(0, _NVW, scan_vreg, (c0, c1))
        pdst = pl.multiple_of(pbase + i * _CE, 128)
        pltpu.sync_copy(d0_v, p0_hbm.at[pl.ds(pdst, _CE)])
        pltpu.sync_copy(d1_v, p1_hbm.at[pl.ds(pdst, _CE)])
        return c0, c1

    lax.fori_loop(0, n_win, window, (zero16, zero16))

    def chunk(c, _):
        crow0 = pl.multiple_of(tr0 + c * _RPC, 16)
        cb0 = tb0 + c * _BPC
        sb0 = sid * _BPT + c * _BPC
        pltpu.sync_copy(ip_hbm.at[pl.ds(crow0, _RPAD)], ip_v)
        pltpu.sync_copy(leg_hbm.at[pl.ds(crow0, _RPC)], leg_v.at[pl.ds(0, _RPC)])
        pltpu.sync_copy(cc_hbm.at[pl.ds(crow0, _RPC)], cc_v.at[pl.ds(0, _RPC)])
        pltpu.sync_copy(pl_hbm.at[pl.ds(cb0 * 16, 256)], pof_v)
        pltpu.sync_copy(hash_hbm.at[pl.ds(cb0 * 16, 256)], hsh_v)

        def mk_ipm1(k, _):
            o = k * 16
            rel = jnp.maximum(ip_v[pl.ds(o, 16)] - 1 - e8, 0)
            ipm1_v[pl.ds(o, 16)] = pbase + rel
            return 0

        lax.fori_loop(0, _NV, mk_ipm1, 0)
        gp0 = pltpu.async_copy(p0_hbm.at[ipm1_v], g0_v, sem)
        gp1 = pltpu.async_copy(p1_hbm.at[ipm1_v], g1_v, sem)
        gp0.wait()
        gp1.wait()

        def batch(bi, _):
            pvec = pof_v[pl.ds(bi * 16, 16)]
            hvec = hsh_v[pl.ds(bi * 16, 16)]
            off = pvec[0] * _N2

            def vrow(jv, _):
                jo = jnp.minimum(jv * 16, _N2 - 16)
                r = bi * _N2 + jo
                ipa = ip_v[pl.ds(r, 16)]
                ipb = ip_v[pl.ds(r + 1, 16)]
                ga0 = jnp.where(ipa == e0, 0, g0_v[pl.ds(r, 16)])
                gb0 = jnp.where(ipb == e0, 0, g0_v[pl.ds(r + 1, 16)])
                ga1 = jnp.where(ipa == e0, 0, g1_v[pl.ds(r, 16)])
                gb1 = jnp.where(ipb == e0, 0, g1_v[pl.ds(r + 1, 16)])
                capd = jnp.where(pvec == 0, ga0 ^ gb0, ga1 ^ gb1)
                ki = (jnp.where(leg_v[pl.ds(r, 16)] != 0, 1, 0)
                      * jnp.where(cc_v[pl.ds(r, 16)] != 0, 1, 0))
                place = ptab_v[pl.ds(off + jo, 16)]
                nh = hvec ^ place ^ jnp.where(ki == 1, capd, 0)
                nh_v[pl.ds(r, 16)] = nh
                lo_v[pl.ds(r, 16)] = zero16
                hi_v[pl.ds(r, 16)] = jnp.full((16,), _N2, jnp.int32)
                ipm1_v[pl.ds(r, 16)] = jnp.full(
                    (16,), (sb0 + bi) * _N2 + (_N2 >> 1), jnp.int32)
                return 0

            lax.fori_loop(0, 23, vrow, 0)
            return 0

        lax.fori_loop(0, _BPC, batch, 0)

        def pad_init(k, _):
            lo_v[pl.ds(k * 16, 16)] = zero16
            hi_v[pl.ds(k * 16, 16)] = jnp.full((16,), _N2, jnp.int32)
            ipm1_v[pl.ds(k * 16, 16)] = jnp.full(
                (16,), (sb0 + _BPC) * _N2 + (_N2 >> 1), jnp.int32)
            return 0

        lax.fori_loop(_RPC // 16, _NV, pad_init, 0)

        def upd(k, _):
            o = k * 16
            lov = lo_v[pl.ds(o, 16)]
            hiv = hi_v[pl.ds(o, 16)]
            mid = (lov + hiv) >> 1
            gi = jnp.where(lov < hiv, 1, 0)
            li = jnp.where(cc_v[pl.ds(o, 16)] < nh_v[pl.ds(o, 16)], 1, 0)
            lov = jnp.where(gi * li == 1, mid + 1, lov)
            hiv = jnp.where(gi * (1 - li) == 1, mid, hiv)
            lo_v[pl.ds(o, 16)] = lov
            hi_v[pl.ds(o, 16)] = hiv
            b_l = lax.div(o + lanes, jnp.int32(_N2))
            hb = (sb0 + b_l) * _N2
            ipm1_v[pl.ds(o, 16)] = hb + jnp.clip((lov + hiv) >> 1, 0, _N2 - 1)
            return 0

        def search_step(_st, _):
            pltpu.async_copy(hists.at[ipm1_v], cc_v, sem).wait()
            lax.fori_loop(0, _NV, upd, 0)
            return 0

        lax.fori_loop(0, 9, search_step, 0)
        pltpu.async_copy(hists.at[ipm1_v], cc_v, sem).wait()

        def fin(k, _):
            o = k * 16
            lov = lo_v[pl.ds(o, 16)]
            f1 = jnp.where(lov < _N2, 1, 0)
            f2 = jnp.where(cc_v[pl.ds(o, 16)] == nh_v[pl.ds(o, 16)], 1, 0)
            l1 = jnp.where(leg_v[pl.ds(o, 16)] != 0, 1, 0)
            hi_v[pl.ds(o, 16)] = l1 * (1 - f1 * f2)
            return 0

        lax.fori_loop(0, _RPC // 16, fin, 0)
        pltpu.sync_copy(nh_v.at[pl.ds(0, _RPC)], nh_hbm.at[pl.ds(crow0, _RPC)])
        pltpu.sync_copy(hi_v.at[pl.ds(0, _RPC)],
                        filt_hbm.at[pl.ds(crow0, _RPC)])
        return 0

    lax.fori_loop(0, _NCH, chunk, 0)


_sc_call = pl.kernel(
    _sc_body,
    out_type=(jax.ShapeDtypeStruct((_ROWS,), jnp.int32),
              jax.ShapeDtypeStruct((_ROWS,), jnp.int32),
              jax.ShapeDtypeStruct((_NW * (_L + 2 * _CE),), jnp.int32),
              jax.ShapeDtypeStruct((_NW * (_L + 2 * _CE),), jnp.int32)),
    mesh=plsc.VectorSubcoreMesh(core_axis_name="c", subcore_axis_name="s"),
    scratch_types=[
        pltpu.VMEM((_RPAD,), jnp.int32),
        pltpu.VMEM((_RPAD,), jnp.int32),
        pltpu.VMEM((_RPAD,), jnp.int32),
        pltpu.VMEM((_RPAD,), jnp.int32),
        pltpu.VMEM((_RPAD,), jnp.int32),
        pltpu.VMEM((_RPAD,), jnp.int32),
        pltpu.VMEM((_RPAD,), jnp.int32),
        pltpu.VMEM((_RPAD,), jnp.int32),
        pltpu.VMEM((_RPAD,), jnp.int32),
        pltpu.VMEM((_CE,), jnp.int32),
        pltpu.VMEM((_CE,), jnp.int32),
        pltpu.VMEM((_CE,), jnp.int32),
        pltpu.VMEM((728,), jnp.int32),
        pltpu.VMEM((256,), jnp.int32),
        pltpu.VMEM((256,), jnp.int32),
        pltpu.VMEM((16,), jnp.int32),
        pltpu.VMEM_SHARED((368,), jnp.int32),
        pltpu.VMEM_SHARED((368,), jnp.int32),
        pltpu.VMEM_SHARED((16 * _NR + 368,), jnp.int32),
        pltpu.SemaphoreType.DMA,
    ],
    name="tensor_board_sc",
)


def kernel(legal_mask, current_player, current_hash, ZposT, cap_indptr,
           cap_indices, can_capture_any, hash_history):
    leg_i = legal_mask.reshape(_B, _N2).astype(jnp.int32).reshape(-1)
    cc_i = can_capture_any.astype(jnp.int32).reshape(-1)
    hist_sorted = jnp.concatenate(
        [jnp.sort(hash_history, axis=1).reshape(-1),
         jnp.zeros((512,), jnp.int32)])
    player = current_player.astype(jnp.int32)
    z0 = ZposT[0]
    zpad = jnp.zeros((7,), jnp.int32)
    d0 = jnp.concatenate([ZposT[2] ^ z0, zpad])
    d1 = jnp.concatenate([ZposT[1] ^ z0, zpad])
    ptab = jnp.concatenate([z0 ^ ZposT[1], z0 ^ ZposT[2],
                            jnp.zeros((6,), jnp.int32)])
    ip_pad = jnp.concatenate(
        [cap_indptr.astype(jnp.int32), jnp.full((127,), _L, jnp.int32)])
    ci_pad = jnp.concatenate(
        [cap_indices.astype(jnp.int32), jnp.zeros((_CE,), jnp.int32)])

    pl_rep = jnp.repeat(player, 16)
    hash_rep = jnp.repeat(current_hash, 16)
    nh, filt, _, _ = _sc_call(ip_pad, ci_pad, leg_i, cc_i, hist_sorted,
                              pl_rep, hash_rep, d0, d1, ptab)
    new_hash = nh.reshape(_B, _N2)
    filtered = filt.astype(jnp.bool_).reshape(_B, 19, 19)
    return filtered, new_hash
